# Initial kernel scaffold; baseline (speedup 1.0000x reference)
import jax
import jax.numpy as jnp
from jax.experimental import pallas as pl


def kernel(x, edge_attr, cartesian_y, edge_index, batch, Wn0, bn0, We0, be0, Wm_bb, bm_bb, Wu_bb, bu_bb, Wbl, bbl, Wm_en, bm_en, Wu_en, bu_en, Wf1, bf1, Wf2, bf2):
    raise NotImplementedError("write your pallas kernel here")


# speedup vs baseline: 3.1664x; 3.1664x over previous
import functools

import jax
import jax.numpy as jnp
from jax import lax
from jax.experimental import pallas as pl
from jax.experimental.pallas import tpu as pltpu
from jax.experimental.pallas import tpu_sc as plsc

_N = 10000
_E = 320000
_H = 128
_EH = 64
_NG = 64
_G = 256
_L = 64

_CH = 128
_NCHUNK = _E // _CH
_NTILES = 32
_CHUNK_ITERS = -(-_NCHUNK // _NTILES)
_NROWP = 10112
_ROWS_PER_TILE = _NROWP // 16
_ZROWS = 128
_ZCHUNKS = (128, 128, 128, 128, 120)

def _sc_mesh():
    return plsc.VectorSubcoreMesh(
        core_axis_name="c", subcore_axis_name="s",
        num_cores=2, num_subcores=16)


@functools.cache
def _sc_edge_diff_kernel():
    return functools.partial(
        pl.kernel,
        out_type=jax.ShapeDtypeStruct((_E, 16), jnp.float32),
        mesh=_sc_mesh(),
        scratch_types=[
            pltpu.VMEM((_CH,), jnp.int32),
            pltpu.VMEM((_CH,), jnp.int32),
            pltpu.VMEM((_CH, 16), jnp.float32),
            pltpu.VMEM((_CH, 16), jnp.float32),
            pltpu.SemaphoreType.DMA,
            pltpu.SemaphoreType.DMA,
        ],
        compiler_params=pltpu.CompilerParams(use_tc_tiling_on_sc=False),
    )(_sc_edge_diff_body)


def _sc_edge_diff(y16, src2d, dst2d):
    return _sc_edge_diff_kernel()(y16, src2d, dst2d)


def _sc_edge_diff_body(y_hbm, s_hbm, d_hbm, out_hbm, sidx, didx, ys, yd,
                       sem1, sem2):
    wid = lax.axis_index("c") * 16 + lax.axis_index("s")

    def step(t, carry):
        cid = wid + t * _NTILES

        @pl.when(cid < _NCHUNK)
        def _():
            pltpu.sync_copy(s_hbm.at[cid], sidx)
            pltpu.sync_copy(d_hbm.at[cid], didx)
            cp1 = pltpu.async_copy(y_hbm.at[sidx], ys, sem1)
            cp2 = pltpu.async_copy(y_hbm.at[didx], yd, sem2)
            cp1.wait()
            cp2.wait()

            def inner(i, c):
                ys[i, :] = ys[i, :] - yd[i, :]
                return c

            lax.fori_loop(0, _CH, inner, 0)
            pltpu.sync_copy(ys, out_hbm.at[pl.ds(cid * _CH, _CH)])

        return carry

    lax.fori_loop(0, _CHUNK_ITERS, step, 0)


@functools.cache
def _sc_message_kernel():
    return functools.partial(
        pl.kernel,
        out_type=jax.ShapeDtypeStruct((2, _NROWP, _H), jnp.float32),
        mesh=_sc_mesh(),
        scratch_types=[
            pltpu.VMEM_SHARED((_NROWP, _H), jnp.float32),
            pltpu.VMEM((_CH,), jnp.int32),
            pltpu.VMEM((1, _CH), jnp.int32),
            pltpu.VMEM((_CH, _H), jnp.float32),
            pltpu.VMEM((_CH, _H), jnp.float32),
            pltpu.VMEM((_ZROWS, _H), jnp.float32),
            pltpu.SemaphoreType.DMA,
        ],
    )(_sc_message_body)


def _sc_message(a, c, src2d, dst2d):
    return _sc_message_kernel()(a, c, src2d, dst2d)


def _sc_message_body(a_hbm, c_hbm, s_hbm, d_hbm, out_hbm,
                     acc, sidx, didx, abuf, cbuf, zbuf, sem):
    core = lax.axis_index("c")
    sid = lax.axis_index("s")
    wid = core * 16 + sid
    base = sid * _ROWS_PER_TILE

    def zrow(i, c):
        r = i // 8
        v = (i % 8) * 16
        zbuf[r, pl.ds(v, 16)] = jnp.zeros((16,), jnp.float32)
        return c

    lax.fori_loop(0, _ZROWS * 8, zrow, 0)
    r0 = 0
    for zc in _ZCHUNKS:
        pltpu.sync_copy(zbuf.at[pl.ds(0, zc)],
                        acc.at[pl.ds(base + r0, zc)])
        r0 += zc
    plsc.subcore_barrier()

    def step(t, carry):
        cid = wid + t * _NTILES

        @pl.when(cid < _NCHUNK)
        def _():
            pltpu.sync_copy(s_hbm.at[cid], sidx)
            pltpu.sync_copy(d_hbm.at[pl.ds(cid, 1)], didx)
            cp = pltpu.async_copy(a_hbm.at[sidx], abuf, sem)
            pltpu.sync_copy(c_hbm.at[pl.ds(cid * _CH, _CH)], cbuf)
            cp.wait()

            def inner(i, c):
                for v in range(_H // 16):
                    sl = pl.ds(v * 16, 16)
                    cbuf[i, sl] = jnp.maximum(abuf[i, sl] + cbuf[i, sl], 0.0)
                return c

            lax.fori_loop(0, _CH, inner, 0)
            pltpu.sync_copy(cbuf, acc.at[didx.at[0]], add=True)

        return carry

    lax.fori_loop(0, _CHUNK_ITERS, step, 0)
    plsc.subcore_barrier()
    r0 = base
    for zc in _ZCHUNKS:
        pltpu.sync_copy(acc.at[pl.ds(r0, zc)],
                        out_hbm.at[core, pl.ds(r0, zc)])
        r0 += zc


_BE = 2560
_BN = 1000
_BT = 400


def _tc_edge_prep(edge_attr, d16, We0, be0, Wbl, bbl):
    step = 10.0 / (_NG - 1)
    coeff = -0.5 / (step * step)

    def body(ea_ref, d_ref, we, be, wbl, bb, e_ref_out, blp_ref_out):
        d = d_ref[...]
        bl2 = jnp.sum(d * d, axis=1, keepdims=True) + 1e-12
        bl = jnp.sqrt(bl2)
        offs = lax.broadcasted_iota(jnp.int32, (1, _NG), 1).astype(
            jnp.float32) * step
        g = jnp.exp(coeff * (bl - offs) ** 2)
        blp_ref_out[...] = jnp.dot(
            g, wbl[...], preferred_element_type=jnp.float32) + bb[...]
        e_ref_out[...] = jnp.maximum(
            jnp.dot(ea_ref[...], we[...], preferred_element_type=jnp.float32)
            + be[...], 0.0)

    return pl.pallas_call(
        body,
        grid=(_E // _BE,),
        in_specs=[
            pl.BlockSpec((_BE, 16), lambda i: (i, 0)),
            pl.BlockSpec((_BE, 16), lambda i: (i, 0)),
            pl.BlockSpec((16, _EH), lambda i: (0, 0)),
            pl.BlockSpec((1, _EH), lambda i: (0, 0)),
            pl.BlockSpec((_NG, _EH), lambda i: (0, 0)),
            pl.BlockSpec((1, _EH), lambda i: (0, 0)),
        ],
        out_specs=[pl.BlockSpec((_BE, _EH), lambda i: (i, 0))] * 2,
        out_shape=[jax.ShapeDtypeStruct((_E, _EH), jnp.float32)] * 2,
    )(edge_attr, d16, We0, be0, Wbl, bbl)


def _tc_c_batched(parts, Ws, bcat, nl):
    np_ = len(parts)

    def body(*refs):
        part_refs = refs[:np_]
        w_refs = refs[np_:2 * np_]
        b = refs[2 * np_]
        outs = refs[2 * np_ + 1:]
        r = b[...]
        for p in range(np_):
            r = r + jnp.dot(part_refs[p][...], w_refs[p][...],
                            preferred_element_type=jnp.float32)
        for l in range(nl):
            outs[l][...] = r[:, l * _H:(l + 1) * _H]

    return pl.pallas_call(
        body,
        grid=(_E // _BE,),
        in_specs=(
            [pl.BlockSpec((_BE, _EH), lambda i: (i, 0))] * np_
            + [pl.BlockSpec((_EH, nl * _H), lambda i: (0, 0))] * np_
            + [pl.BlockSpec((1, nl * _H), lambda i: (0, 0))]
        ),
        out_specs=[pl.BlockSpec((_BE, _H), lambda i: (i, 0))] * nl,
        out_shape=[jax.ShapeDtypeStruct((_E, _H), jnp.float32)] * nl,
    )(*parts, *Ws, bcat)


def _tc_node0(x, Wn0, bn0, Wt):
    def body(x_ref, wn, bn, wt, h_out, a_out):
        h = jnp.maximum(
            jnp.dot(x_ref[...], wn[...], preferred_element_type=jnp.float32)
            + bn[...], 0.0)
        h_out[...] = h
        a_out[...] = jnp.dot(h, wt[...], preferred_element_type=jnp.float32)

    return pl.pallas_call(
        body,
        grid=(_N // _BN,),
        in_specs=[
            pl.BlockSpec((_BN, _H), lambda i: (i, 0)),
            pl.BlockSpec((_H, _H), lambda i: (0, 0)),
            pl.BlockSpec((1, _H), lambda i: (0, 0)),
            pl.BlockSpec((_H, _H), lambda i: (0, 0)),
        ],
        out_specs=[pl.BlockSpec((_BN, _H), lambda i: (i, 0))] * 2,
        out_shape=[jax.ShapeDtypeStruct((_N, _H), jnp.float32)] * 2,
    )(x, Wn0, bn0, Wt)


def _tc_update(h, P, Wu, bu, Wt):
    has_a = Wt is not None

    def body(h_ref, p_ref, wu, bu_ref, *rest):
        agg = p_ref[0] + p_ref[1]
        cat = jnp.concatenate([h_ref[...], agg], axis=1)
        hn = jnp.maximum(
            jnp.dot(cat, wu[...], preferred_element_type=jnp.float32)
            + bu_ref[...], 0.0)
        if has_a:
            wt, h_out, a_out = rest
            h_out[...] = hn
            a_out[...] = jnp.dot(hn, wt[...],
                                 preferred_element_type=jnp.float32)
        else:
            (h_out,) = rest
            h_out[...] = hn

    in_specs = [
        pl.BlockSpec((_BN, _H), lambda i: (i, 0)),
        pl.BlockSpec((2, _BN, _H), lambda i: (0, i, 0)),
        pl.BlockSpec((2 * _H, _H), lambda i: (0, 0)),
        pl.BlockSpec((1, _H), lambda i: (0, 0)),
    ]
    args = [h, P, Wu, bu]
    nouts = 1
    if has_a:
        in_specs.append(pl.BlockSpec((_H, _H), lambda i: (0, 0)))
        args.append(Wt)
        nouts = 2
    out = pl.pallas_call(
        body,
        grid=(_N // _BN,),
        in_specs=in_specs,
        out_specs=[pl.BlockSpec((_BN, _H), lambda i: (i, 0))] * nouts,
        out_shape=[jax.ShapeDtypeStruct((_N, _H), jnp.float32)] * nouts,
    )(*args)
    return out if has_a else (out[0], None)


def _tc_tail(h, batch_row3, batch_col, Wf1, bf1, Wf2, bf2):
    nb = _N // _BT

    def body(h_ref, br_ref, bc_ref, wf1, bf1_ref, wf2, bf2_ref, out_ref,
             sums, sumsq, counts):
        p = pl.program_id(0)

        @pl.when(jnp.logical_and(p == 0, pl.program_id(1) == 0))
        def _():
            sums[...] = jnp.zeros_like(sums)
            sumsq[...] = jnp.zeros_like(sumsq)
            counts[...] = jnp.zeros_like(counts)

        h_blk = h_ref[...]

        @pl.when(p == 0)
        def _():
            oh = (lax.broadcasted_iota(jnp.int32, (_G, _BT), 0)
                  == br_ref[0]).astype(jnp.float32)
            sums[...] += jnp.dot(oh, h_blk, preferred_element_type=jnp.float32,
                                 precision=lax.Precision.HIGHEST)
            sumsq[...] += jnp.dot(oh, h_blk * h_blk,
                                  preferred_element_type=jnp.float32,
                                  precision=lax.Precision.HIGHEST)
            counts[...] += jnp.sum(oh, axis=1, keepdims=True)
            out_ref[...] = jnp.zeros_like(out_ref)

        @pl.when(p == 1)
        def _():
            cnt = jnp.maximum(counts[...], 1.0)
            mean = sums[...] / cnt
            var = sumsq[...] / cnt - mean * mean
            oh2 = (lax.broadcasted_iota(jnp.int32, (_BT, _G), 1)
                   == bc_ref[...]).astype(jnp.float32)
            mean_n = jnp.dot(oh2, mean, preferred_element_type=jnp.float32,
                             precision=lax.Precision.HIGHEST)
            var_n = jnp.dot(oh2, var, preferred_element_type=jnp.float32,
                            precision=lax.Precision.HIGHEST)
            hn = (h_blk - mean_n) / jnp.sqrt(
                jnp.maximum(var_n, 0.0) + 1e-5)
            z = jnp.dot(hn, wf1[...], preferred_element_type=jnp.float32) \
                + bf1_ref[...]
            z = jnp.where(z > 0, z, 0.01 * z)
            out_ref[...] = jnp.dot(z, wf2[...],
                                   preferred_element_type=jnp.float32) \
                + bf2_ref[...]

    return pl.pallas_call(
        body,
        grid=(2, nb),
        in_specs=[
            pl.BlockSpec((_BT, _H), lambda p, b: (b, 0)),
            pl.BlockSpec((1, 1, _BT), lambda p, b: (b, 0, 0)),
            pl.BlockSpec((_BT, 1), lambda p, b: (b, 0)),
            pl.BlockSpec((_H, _H // 2), lambda p, b: (0, 0)),
            pl.BlockSpec((1, _H // 2), lambda p, b: (0, 0)),
            pl.BlockSpec((_H // 2, _L), lambda p, b: (0, 0)),
            pl.BlockSpec((1, _L), lambda p, b: (0, 0)),
        ],
        out_specs=pl.BlockSpec((_BT, _L), lambda p, b: (b, 0)),
        out_shape=jax.ShapeDtypeStruct((_N, _L), jnp.float32),
        scratch_shapes=[
            pltpu.VMEM((_G, _H), jnp.float32),
            pltpu.VMEM((_G, _H), jnp.float32),
            pltpu.VMEM((_G, 1), jnp.float32),
        ],
    )(h, batch_row3, batch_col, Wf1, bf1, Wf2, bf2)


def kernel(x, edge_attr, cartesian_y, edge_index, batch,
           Wn0, bn0, We0, be0, Wm_bb, bm_bb, Wu_bb, bu_bb, Wbl, bbl,
           Wm_en, bm_en, Wu_en, bu_en, Wf1, bf1, Wf2, bf2):
    src2d = edge_index[0].astype(jnp.int32).reshape(_NCHUNK, _CH)
    dst2d = edge_index[1].astype(jnp.int32).reshape(_NCHUNK, _CH)
    y16 = jnp.pad(cartesian_y, ((0, 0), (0, 13)))

    d16 = _sc_edge_diff(y16, src2d, dst2d)
    e_feat, blp = _tc_edge_prep(edge_attr, d16, We0, be0.reshape(1, -1),
                                Wbl, bbl.reshape(1, -1))

    Wc_bb = jnp.concatenate([Wm_bb[l][_H:, :] for l in range(6)], axis=1)
    C_bb = _tc_c_batched([e_feat], [Wc_bb], bm_bb.reshape(1, -1), 6)
    Wc_en_e = jnp.concatenate(
        [Wm_en[l][_H:_H + _EH, :] for l in range(3)], axis=1)
    Wc_en_b = jnp.concatenate(
        [Wm_en[l][_H + _EH:, :] for l in range(3)], axis=1)
    C_en = _tc_c_batched([e_feat, blp], [Wc_en_e, Wc_en_b],
                         bm_en.reshape(1, -1), 3)

    Ws_top = [Wm_bb[l][:_H] for l in range(6)] + [Wm_en[l][:_H] for l in range(3)]
    Wus = [Wu_bb[l] for l in range(6)] + [Wu_en[l] for l in range(3)]
    bus = ([bu_bb[l].reshape(1, -1) for l in range(6)]
           + [bu_en[l].reshape(1, -1) for l in range(3)])
    Cs = list(C_bb) + list(C_en)

    h, A = _tc_node0(x, Wn0, bn0.reshape(1, -1), Ws_top[0])
    for l in range(9):
        P = _sc_message(A, Cs[l], src2d, dst2d)
        Wt = Ws_top[l + 1] if l < 8 else None
        h, A = _tc_update(h, P, Wus[l], bus[l], Wt)

    batch_i32 = batch.astype(jnp.int32)
    batch_row3 = batch_i32.reshape(_N // _BT, 1, _BT)
    batch_col = batch_i32.reshape(_N, 1)
    return _tc_tail(h, batch_row3, batch_col,
                    Wf1, bf1.reshape(1, -1), Wf2, bf2.reshape(1, -1))
